# baseline (speedup 1.0000x reference)
import functools

import jax
import jax.numpy as jnp
from jax import lax
from jax.experimental import pallas as pl
from jax.experimental.pallas import tpu as pltpu
from jax.experimental.pallas import tpu_sc as plsc

T = 4096
D = 1024
E = 8
NRE = 7
H = 4096
ALPHA = 2.0

BT = 128
BTLOG = 7
NBR = 38
TP = NBR * BT
BH = 1024
NH = H // BH

NC, NS = 2, 16
NW = NC * NS
TW = T // NW
NBID = 48
SG = 128


RBT = 512
NTB = T // RBT


def _router_body(x_ref, w_ref, eid_ref, scale_ref, cnt_ref):
    lg = jnp.dot(x_ref[...], w_ref[...], preferred_element_type=jnp.float32)
    m = jnp.max(lg, axis=1, keepdims=True)
    col = lax.broadcasted_iota(jnp.int32, lg.shape, 1)
    amax = jnp.min(jnp.where(lg == m, col, E), axis=1)
    p = 1.0 / jnp.sum(jnp.exp(lg - m), axis=1)
    routed = amax < NRE
    eid = jnp.where(routed, amax, 0)
    eid_ref[0, 0, :] = eid
    scale_ref[0, 0, :] = jnp.where(routed, ALPHA * p, 0.0)
    cols16 = lax.broadcasted_iota(jnp.int32, (RBT, 16), 1)
    oh = (eid[:, None] == cols16).astype(jnp.int32)
    for c2 in range(RBT // TW):
        cnt_ref[0, c2, :] = jnp.sum(oh[c2 * TW:(c2 + 1) * TW], axis=0)


def _router(x, router_w):
    eid, scale, cnt = pl.pallas_call(
        _router_body,
        grid=(NTB,),
        in_specs=[
            pl.BlockSpec((RBT, D), lambda i: (i, 0)),
            pl.BlockSpec((D, E), lambda i: (0, 0)),
        ],
        out_specs=[
            pl.BlockSpec((1, 1, RBT), lambda i: (i, 0, 0)),
            pl.BlockSpec((1, 1, RBT), lambda i: (i, 0, 0)),
            pl.BlockSpec((1, RBT // TW, 16), lambda i: (i, 0, 0)),
        ],
        out_shape=[
            jax.ShapeDtypeStruct((NTB, 1, RBT), jnp.int32),
            jax.ShapeDtypeStruct((NTB, 1, RBT), jnp.float32),
            jax.ShapeDtypeStruct((NTB, RBT // TW, 16), jnp.int32),
        ],
    )(x, router_w)
    return eid.reshape(T), scale.reshape(T), cnt.reshape(NW * 16)


def _sort_scatter(eid, scale, cnt, x):
    mesh = plsc.VectorSubcoreMesh(core_axis_name="c", subcore_axis_name="s",
                                  num_cores=NC, num_subcores=NS)

    @functools.partial(
        pl.kernel,
        mesh=mesh,
        compiler_params=pltpu.CompilerParams(needs_layout_passes=False),
        out_type=(
            jax.ShapeDtypeStruct((T,), jnp.int32),
            jax.ShapeDtypeStruct((NBID,), jnp.int32),
            jax.ShapeDtypeStruct((TP, D), jnp.float32),
            jax.ShapeDtypeStruct((TP, SG), jnp.float32),
        ),
        scratch_types=[
            pltpu.VMEM((NW * 16,), jnp.int32),
            pltpu.VMEM((TW,), jnp.int32),
            pltpu.VMEM((TW,), jnp.float32),
            pltpu.VMEM((NBID,), jnp.int32),
            pltpu.VMEM((TW,), jnp.int32),
            pltpu.VMEM((64,), jnp.int32),
            pltpu.VMEM((64,), jnp.int32),
            pltpu.VMEM((64, D), jnp.float32),
            pltpu.VMEM((64, SG), jnp.float32),
            pltpu.SemaphoreType.DMA,
        ],
    )
    def k(eid_hbm, scale_hbm, cnt_hbm, x_hbm,
          dest_hbm, beid_hbm, xs_hbm, ss_hbm,
          cntv, eidv, scalev, beidv, destv, idxA, idxB, xbuf, sbuf, sem):
        wid = lax.axis_index("s") * NC + lax.axis_index("c")
        base = wid * TW
        lanes = lax.iota(jnp.int32, 16)

        pltpu.sync_copy(cnt_hbm, cntv)
        pltpu.sync_copy(eid_hbm.at[pl.ds(base, TW)], eidv)
        pltpu.sync_copy(scale_hbm.at[pl.ds(base, TW)], scalev)

        tot = jnp.zeros((16,), jnp.int32)
        pref = jnp.zeros((16,), jnp.int32)
        for w in range(NW):
            row = cntv[pl.ds(w * 16, 16)]
            tot = tot + row
            pref = pref + jnp.where(w < wid, row, jnp.zeros((16,), jnp.int32))
        nblk = (tot + (BT - 1)) >> BTLOG
        blk_incl = plsc.cumsum(nblk)
        blk_excl = blk_incl - nblk
        wbase = blk_excl * BT + pref

        bused = blk_incl[NRE - 1]
        for c2 in range(NBID // 16):
            bv = lax.iota(jnp.int32, 16) + c2 * 16
            acc = jnp.zeros((16,), jnp.int32)
            for e in range(NRE):
                s = blk_excl[e]
                n = nblk[e]
                acc = jnp.where((bv >= s) & (bv < s + n), e, acc)
            acc = jnp.where(bv < bused, acc, -1)
            beidv[pl.ds(c2 * 16, 16)] = acc

        @pl.when(wid == 0)
        def _():
            pltpu.sync_copy(beidv, beid_hbm)

        run = wbase
        for j in range(TW // 16):
            v = eidv[pl.ds(j * 16, 16)]
            dest16 = jnp.zeros((16,), jnp.int32)
            for e in range(NRE):
                m = v == e
                csum = plsc.cumsum(jnp.where(m, 1, 0))
                r_es = run[e]
                dest16 = jnp.where(m, r_es + csum - 1, dest16)
                run = run + jnp.where(lanes == e, csum[15], 0)
            destv[pl.ds(j * 16, 16)] = dest16
            half = idxA if j < 4 else idxB
            half[pl.ds((j % 4) * 16, 16)] = dest16
        pltpu.sync_copy(destv, dest_hbm.at[pl.ds(base, TW)])

        for half, idx in ((0, idxA), (1, idxB)):
            hb = base + half * 64
            for q in range(4):
                s16 = scalev[pl.ds(half * 64 + q * 16, 16)]
                for r in range(16):
                    sbuf[q * 16 + r, pl.ds(0, 16)] = jnp.full(
                        (16,), s16[r], jnp.float32)
            pltpu.sync_copy(x_hbm.at[pl.ds(hb, 64)], xbuf)
            pltpu.async_copy(xbuf, xs_hbm.at[idx], sem).wait()
            pltpu.async_copy(sbuf, ss_hbm.at[idx], sem).wait()

    return k(eid, scale, cnt, x)


def _mlp_body(beid_s, xs_ref, ss_ref, eu, eg, ed, su, sg, sd, out_ref,
              acc_ref):
    h = pl.program_id(0)
    b = pl.program_id(1)

    @pl.when(beid_s[b] >= 0)
    def _():
        xb = xs_ref[...].astype(jnp.bfloat16)
        up = jnp.dot(xb, eu[0], preferred_element_type=jnp.float32)
        gt = jnp.dot(xb, eg[0], preferred_element_type=jnp.float32)
        a = (up * (gt * jax.nn.sigmoid(gt))).astype(jnp.bfloat16)
        part_r = jnp.dot(a, ed[0], preferred_element_type=jnp.float32)
        ups = jnp.dot(xb, su[...], preferred_element_type=jnp.float32)
        gts = jnp.dot(xb, sg[...], preferred_element_type=jnp.float32)
        a_s = (ups * (gts * jax.nn.sigmoid(gts))).astype(jnp.bfloat16)
        part_s = jnp.dot(a_s, sd[...], preferred_element_type=jnp.float32)
        sc = ss_ref[:, 0:1]
        part = sc * part_r + part_s
        sl = pl.ds(b * BT, BT)

        @pl.when(h == 0)
        def _():
            acc_ref[sl, :] = part

        @pl.when(h > 0)
        def _():
            acc_ref[sl, :] = acc_ref[sl, :] + part

        @pl.when(h == NH - 1)
        def _():
            out_ref[...] = acc_ref[sl, :]


def _grouped_mlp(beid, xs, ss, eu, eg, ed, su, sg, sd):
    grid_spec = pltpu.PrefetchScalarGridSpec(
        num_scalar_prefetch=1,
        grid=(NH, NBR),
        in_specs=[
            pl.BlockSpec((BT, D), lambda h, b, beid_s: (b, 0)),
            pl.BlockSpec((BT, SG), lambda h, b, beid_s: (b, 0)),
            pl.BlockSpec((1, D, BH),
                         lambda h, b, beid_s: (jnp.where(beid_s[b] >= 0, beid_s[b], NRE - 1), 0, h)),
            pl.BlockSpec((1, D, BH),
                         lambda h, b, beid_s: (jnp.where(beid_s[b] >= 0, beid_s[b], NRE - 1), 0, h)),
            pl.BlockSpec((1, BH, D),
                         lambda h, b, beid_s: (jnp.where(beid_s[b] >= 0, beid_s[b], NRE - 1), h, 0)),
            pl.BlockSpec((D, BH), lambda h, b, beid_s: (0, h)),
            pl.BlockSpec((D, BH), lambda h, b, beid_s: (0, h)),
            pl.BlockSpec((BH, D), lambda h, b, beid_s: (h, 0)),
        ],
        out_specs=pl.BlockSpec((BT, D), lambda h, b, beid_s: (b, 0)),
        scratch_shapes=[pltpu.VMEM((TP, D), jnp.float32)],
    )
    return pl.pallas_call(
        _mlp_body,
        grid_spec=grid_spec,
        out_shape=jax.ShapeDtypeStruct((TP, D), jnp.float32),
        compiler_params=pltpu.CompilerParams(
            dimension_semantics=("arbitrary", "arbitrary")),
    )(beid, xs, ss, eu, eg, ed, su, sg, sd)


def _gather_out(rows, dest):
    mesh = plsc.VectorSubcoreMesh(core_axis_name="c", subcore_axis_name="s",
                                  num_cores=NC, num_subcores=NS)

    @functools.partial(
        pl.kernel,
        mesh=mesh,
        compiler_params=pltpu.CompilerParams(needs_layout_passes=False),
        out_type=jax.ShapeDtypeStruct((T, D), jnp.float32),
        scratch_types=[
            pltpu.VMEM((TW,), jnp.int32),
            pltpu.VMEM((64,), jnp.int32),
            pltpu.VMEM((64, D), jnp.float32),
            pltpu.SemaphoreType.DMA,
        ],
    )
    def k(rows_hbm, dest_hbm, out_hbm, destv, idx, buf, sem):
        wid = lax.axis_index("s") * NC + lax.axis_index("c")
        base = wid * TW
        pltpu.sync_copy(dest_hbm.at[pl.ds(base, TW)], destv)
        for half in range(2):
            for q in range(4):
                idx[pl.ds(q * 16, 16)] = destv[pl.ds(half * 64 + q * 16, 16)]
            pltpu.async_copy(rows_hbm.at[idx], buf, sem).wait()
            pltpu.sync_copy(buf, out_hbm.at[pl.ds(base + half * 64, 64)])

    return k(rows, dest)


def kernel(x, router_w, expert_up, expert_gate, expert_down,
           shared_up, shared_gate, shared_down):
    bf = jnp.bfloat16
    eid, scale, cnt = _router(x, router_w)
    dest, beid, xs, ss = _sort_scatter(eid, scale, cnt, x)
    rows = _grouped_mlp(beid, xs, ss,
                        expert_up.astype(bf), expert_gate.astype(bf),
                        expert_down.astype(bf), shared_up.astype(bf),
                        shared_gate.astype(bf), shared_down.astype(bf))
    return _gather_out(rows, dest)


# speedup vs baseline: 1.2472x; 1.2472x over previous
import functools

import jax
import jax.numpy as jnp
from jax import lax
from jax.experimental import pallas as pl
from jax.experimental.pallas import tpu as pltpu
from jax.experimental.pallas import tpu_sc as plsc

T = 4096
D = 1024
E = 8
NRE = 7
H = 4096
ALPHA = 2.0

BT = 128
BTLOG = 7
NBR = 38
TP = NBR * BT
BH = 1024
NH = H // BH

NC, NS = 2, 16
NW = NC * NS
TW = T // NW
NBID = 48
SG = 128


RBT = 512
NTB = T // RBT


def _router_body(x_ref, w_ref, eid_ref, scale_ref, cnt_ref):
    lg = jnp.dot(x_ref[...], w_ref[...], preferred_element_type=jnp.float32)
    m = jnp.max(lg, axis=1, keepdims=True)
    col = lax.broadcasted_iota(jnp.int32, lg.shape, 1)
    amax = jnp.min(jnp.where(lg == m, col, E), axis=1)
    p = 1.0 / jnp.sum(jnp.exp(lg - m), axis=1)
    routed = amax < NRE
    eid = jnp.where(routed, amax, 0)
    eid_ref[0, 0, :] = eid
    scale_ref[0, 0, :] = jnp.where(routed, ALPHA * p, 0.0)
    cols16 = lax.broadcasted_iota(jnp.int32, (RBT, 16), 1)
    oh = (eid[:, None] == cols16).astype(jnp.int32)
    for c2 in range(RBT // TW):
        cnt_ref[0, c2, :] = jnp.sum(oh[c2 * TW:(c2 + 1) * TW], axis=0)


def _router(x, router_w):
    eid, scale, cnt = pl.pallas_call(
        _router_body,
        grid=(NTB,),
        in_specs=[
            pl.BlockSpec((RBT, D), lambda i: (i, 0)),
            pl.BlockSpec((D, E), lambda i: (0, 0)),
        ],
        out_specs=[
            pl.BlockSpec((1, 1, RBT), lambda i: (i, 0, 0)),
            pl.BlockSpec((1, 1, RBT), lambda i: (i, 0, 0)),
            pl.BlockSpec((1, RBT // TW, 16), lambda i: (i, 0, 0)),
        ],
        out_shape=[
            jax.ShapeDtypeStruct((NTB, 1, RBT), jnp.int32),
            jax.ShapeDtypeStruct((NTB, 1, RBT), jnp.float32),
            jax.ShapeDtypeStruct((NTB, RBT // TW, 16), jnp.int32),
        ],
    )(x, router_w)
    return eid.reshape(T), scale.reshape(T), cnt.reshape(NW * 16)


def _sort_scatter(eid, scale, cnt, x):
    mesh = plsc.VectorSubcoreMesh(core_axis_name="c", subcore_axis_name="s",
                                  num_cores=NC, num_subcores=NS)

    @functools.partial(
        pl.kernel,
        mesh=mesh,
        compiler_params=pltpu.CompilerParams(needs_layout_passes=False),
        out_type=(
            jax.ShapeDtypeStruct((T,), jnp.int32),
            jax.ShapeDtypeStruct((NBID,), jnp.int32),
            jax.ShapeDtypeStruct((TP, D), jnp.float32),
            jax.ShapeDtypeStruct((TP, SG), jnp.float32),
        ),
        scratch_types=[
            pltpu.VMEM((NW * 16,), jnp.int32),
            pltpu.VMEM((TW,), jnp.int32),
            pltpu.VMEM((TW,), jnp.float32),
            pltpu.VMEM((NBID,), jnp.int32),
            pltpu.VMEM((TW,), jnp.int32),
            pltpu.VMEM((64,), jnp.int32),
            pltpu.VMEM((64,), jnp.int32),
            pltpu.VMEM((64, D), jnp.float32),
            pltpu.VMEM((64, SG), jnp.float32),
            pltpu.SemaphoreType.DMA,
        ],
    )
    def k(eid_hbm, scale_hbm, cnt_hbm, x_hbm,
          dest_hbm, beid_hbm, xs_hbm, ss_hbm,
          cntv, eidv, scalev, beidv, destv, idxA, idxB, xbuf, sbuf, sem):
        wid = lax.axis_index("s") * NC + lax.axis_index("c")
        base = wid * TW
        lanes = lax.iota(jnp.int32, 16)

        pltpu.sync_copy(cnt_hbm, cntv)
        pltpu.sync_copy(eid_hbm.at[pl.ds(base, TW)], eidv)
        pltpu.sync_copy(scale_hbm.at[pl.ds(base, TW)], scalev)

        tot = jnp.zeros((16,), jnp.int32)
        pref = jnp.zeros((16,), jnp.int32)
        for w in range(NW):
            row = cntv[pl.ds(w * 16, 16)]
            tot = tot + row
            pref = pref + jnp.where(w < wid, row, jnp.zeros((16,), jnp.int32))
        nblk = (tot + (BT - 1)) >> BTLOG
        blk_incl = plsc.cumsum(nblk)
        blk_excl = blk_incl - nblk
        wbase = blk_excl * BT + pref

        bused = blk_incl[NRE - 1]
        for c2 in range(NBID // 16):
            bv = lax.iota(jnp.int32, 16) + c2 * 16
            acc = jnp.zeros((16,), jnp.int32)
            for e in range(NRE):
                s = blk_excl[e]
                n = nblk[e]
                acc = jnp.where((bv >= s) & (bv < s + n), e, acc)
            acc = jnp.where(bv < bused, acc, -1)
            beidv[pl.ds(c2 * 16, 16)] = acc

        @pl.when(wid == 0)
        def _():
            pltpu.sync_copy(beidv, beid_hbm)

        run = wbase
        for j in range(TW // 16):
            v = eidv[pl.ds(j * 16, 16)]
            dest16 = jnp.zeros((16,), jnp.int32)
            for e in range(NRE):
                m = v == e
                csum = plsc.cumsum(jnp.where(m, 1, 0))
                r_es = run[e]
                dest16 = jnp.where(m, r_es + csum - 1, dest16)
                run = run + jnp.where(lanes == e, csum[15], 0)
            destv[pl.ds(j * 16, 16)] = dest16
            half = idxA if j < 4 else idxB
            half[pl.ds((j % 4) * 16, 16)] = dest16
        pltpu.sync_copy(destv, dest_hbm.at[pl.ds(base, TW)])

        for half, idx in ((0, idxA), (1, idxB)):
            hb = base + half * 64
            for q in range(4):
                s16 = scalev[pl.ds(half * 64 + q * 16, 16)]
                for r in range(16):
                    sbuf[q * 16 + r, pl.ds(0, 16)] = jnp.full(
                        (16,), s16[r], jnp.float32)
            pltpu.sync_copy(x_hbm.at[pl.ds(hb, 64)], xbuf)
            pltpu.async_copy(xbuf, xs_hbm.at[idx], sem).wait()
            pltpu.async_copy(sbuf, ss_hbm.at[idx], sem).wait()

    return k(eid, scale, cnt, x)


def _mlp_body(beid_s, xs_ref, ss_ref, eu, eg, ed, su, sg, sd, out_ref,
              acc_ref):
    h = pl.program_id(0)
    b = pl.program_id(1)

    @pl.when(beid_s[b] >= 0)
    def _():
        xb = xs_ref[...]
        up = jnp.dot(xb, eu[0], preferred_element_type=jnp.float32)
        gt = jnp.dot(xb, eg[0], preferred_element_type=jnp.float32)
        a = up * (gt * jax.nn.sigmoid(gt))
        part_r = jnp.dot(a, ed[0], preferred_element_type=jnp.float32)
        ups = jnp.dot(xb, su[...], preferred_element_type=jnp.float32)
        gts = jnp.dot(xb, sg[...], preferred_element_type=jnp.float32)
        a_s = ups * (gts * jax.nn.sigmoid(gts))
        part_s = jnp.dot(a_s, sd[...], preferred_element_type=jnp.float32)
        sc = ss_ref[:, 0:1]
        part = sc * part_r + part_s
        sl = pl.ds(b * BT, BT)

        @pl.when(h == 0)
        def _():
            acc_ref[sl, :] = part.astype(jnp.bfloat16)

        @pl.when((h > 0) & (h < NH - 1))
        def _():
            acc_ref[sl, :] = (acc_ref[sl, :].astype(jnp.float32)
                              + part).astype(jnp.bfloat16)

        @pl.when(h == NH - 1)
        def _():
            out_ref[...] = acc_ref[sl, :].astype(jnp.float32) + part


def _grouped_mlp(beid, xs, ss, eu, eg, ed, su, sg, sd):
    grid_spec = pltpu.PrefetchScalarGridSpec(
        num_scalar_prefetch=1,
        grid=(NH, NBR),
        in_specs=[
            pl.BlockSpec((BT, D), lambda h, b, beid_s: (b, 0)),
            pl.BlockSpec((BT, SG), lambda h, b, beid_s: (b, 0)),
            pl.BlockSpec((1, D, BH),
                         lambda h, b, beid_s: (jnp.where(beid_s[b] >= 0, beid_s[b], NRE - 1), 0, h)),
            pl.BlockSpec((1, D, BH),
                         lambda h, b, beid_s: (jnp.where(beid_s[b] >= 0, beid_s[b], NRE - 1), 0, h)),
            pl.BlockSpec((1, BH, D),
                         lambda h, b, beid_s: (jnp.where(beid_s[b] >= 0, beid_s[b], NRE - 1), h, 0)),
            pl.BlockSpec((D, BH), lambda h, b, beid_s: (0, h)),
            pl.BlockSpec((D, BH), lambda h, b, beid_s: (0, h)),
            pl.BlockSpec((BH, D), lambda h, b, beid_s: (h, 0)),
        ],
        out_specs=pl.BlockSpec((BT, D), lambda h, b, beid_s: (b, 0)),
        scratch_shapes=[pltpu.VMEM((TP, D), jnp.bfloat16)],
    )
    return pl.pallas_call(
        _mlp_body,
        grid_spec=grid_spec,
        out_shape=jax.ShapeDtypeStruct((TP, D), jnp.float32),
        compiler_params=pltpu.CompilerParams(
            dimension_semantics=("arbitrary", "arbitrary"),
            vmem_limit_bytes=64 * 1024 * 1024),
    )(beid, xs, ss, eu, eg, ed, su, sg, sd)


def _gather_out(rows, dest):
    mesh = plsc.VectorSubcoreMesh(core_axis_name="c", subcore_axis_name="s",
                                  num_cores=NC, num_subcores=NS)

    @functools.partial(
        pl.kernel,
        mesh=mesh,
        compiler_params=pltpu.CompilerParams(needs_layout_passes=False),
        out_type=jax.ShapeDtypeStruct((T, D), jnp.float32),
        scratch_types=[
            pltpu.VMEM((TW,), jnp.int32),
            pltpu.VMEM((64,), jnp.int32),
            pltpu.VMEM((64, D), jnp.float32),
            pltpu.SemaphoreType.DMA,
        ],
    )
    def k(rows_hbm, dest_hbm, out_hbm, destv, idx, buf, sem):
        wid = lax.axis_index("s") * NC + lax.axis_index("c")
        base = wid * TW
        pltpu.sync_copy(dest_hbm.at[pl.ds(base, TW)], destv)
        for half in range(2):
            for q in range(4):
                idx[pl.ds(q * 16, 16)] = destv[pl.ds(half * 64 + q * 16, 16)]
            pltpu.async_copy(rows_hbm.at[idx], buf, sem).wait()
            pltpu.sync_copy(buf, out_hbm.at[pl.ds(base + half * 64, 64)])

    return k(rows, dest)


def kernel(x, router_w, expert_up, expert_gate, expert_down,
           shared_up, shared_gate, shared_down):
    eid, scale, cnt = _router(x, router_w)
    dest, beid, xs, ss = _sort_scatter(eid, scale, cnt, x)
    rows = _grouped_mlp(beid, xs, ss, expert_up, expert_gate, expert_down,
                        shared_up, shared_gate, shared_down)
    return _gather_out(rows, dest)


# speedup vs baseline: 1.4414x; 1.1557x over previous
import functools

import jax
import jax.numpy as jnp
from jax import lax
from jax.experimental import pallas as pl
from jax.experimental.pallas import tpu as pltpu
from jax.experimental.pallas import tpu_sc as plsc

T = 4096
D = 1024
E = 8
NRE = 7
H = 4096
ALPHA = 2.0

BT = 128
BTLOG = 7
NBR = 38
TP = NBR * BT
BH = 1024
NH = H // BH

NC, NS = 2, 16
NW = NC * NS
TW = T // NW
NBID = 48
SG = 128


RBT = 512
NTB = T // RBT


def _router_body(x_ref, w_ref, eid_ref, scale_ref, cnt_ref):
    lg = jnp.dot(x_ref[...], w_ref[...], preferred_element_type=jnp.float32)
    m = jnp.max(lg, axis=1, keepdims=True)
    col = lax.broadcasted_iota(jnp.int32, lg.shape, 1)
    amax = jnp.min(jnp.where(lg == m, col, E), axis=1)
    p = 1.0 / jnp.sum(jnp.exp(lg - m), axis=1)
    routed = amax < NRE
    eid = jnp.where(routed, amax, 0)
    eid_ref[0, 0, :] = eid
    scale_ref[0, 0, :] = jnp.where(routed, ALPHA * p, 0.0)
    cols16 = lax.broadcasted_iota(jnp.int32, (RBT, 16), 1)
    oh = (eid[:, None] == cols16).astype(jnp.int32)
    for c2 in range(RBT // TW):
        cnt_ref[0, c2, :] = jnp.sum(oh[c2 * TW:(c2 + 1) * TW], axis=0)


def _router(x, router_w):
    eid, scale, cnt = pl.pallas_call(
        _router_body,
        grid=(NTB,),
        in_specs=[
            pl.BlockSpec((RBT, D), lambda i: (i, 0)),
            pl.BlockSpec((D, E), lambda i: (0, 0)),
        ],
        out_specs=[
            pl.BlockSpec((1, 1, RBT), lambda i: (i, 0, 0)),
            pl.BlockSpec((1, 1, RBT), lambda i: (i, 0, 0)),
            pl.BlockSpec((1, RBT // TW, 16), lambda i: (i, 0, 0)),
        ],
        out_shape=[
            jax.ShapeDtypeStruct((NTB, 1, RBT), jnp.int32),
            jax.ShapeDtypeStruct((NTB, 1, RBT), jnp.float32),
            jax.ShapeDtypeStruct((NTB, RBT // TW, 16), jnp.int32),
        ],
    )(x, router_w)
    return eid.reshape(T), scale.reshape(T), cnt.reshape(NW * 16)


def _sort_scatter(eid, scale, cnt, x):
    mesh = plsc.VectorSubcoreMesh(core_axis_name="c", subcore_axis_name="s",
                                  num_cores=NC, num_subcores=NS)

    @functools.partial(
        pl.kernel,
        mesh=mesh,
        compiler_params=pltpu.CompilerParams(needs_layout_passes=False),
        out_type=(
            jax.ShapeDtypeStruct((T,), jnp.int32),
            jax.ShapeDtypeStruct((NBID,), jnp.int32),
            jax.ShapeDtypeStruct((NBID + 16,), jnp.int32),
            jax.ShapeDtypeStruct((TP, D), jnp.float32),
            jax.ShapeDtypeStruct((TP, SG), jnp.float32),
        ),
        scratch_types=[
            pltpu.VMEM((NW * 16,), jnp.int32),
            pltpu.VMEM((TW,), jnp.int32),
            pltpu.VMEM((TW,), jnp.float32),
            pltpu.VMEM((NBID,), jnp.int32),
            pltpu.VMEM((NBID + 16,), jnp.int32),
            pltpu.VMEM((TW,), jnp.int32),
            pltpu.VMEM((64,), jnp.int32),
            pltpu.VMEM((64,), jnp.int32),
            pltpu.VMEM((64, D), jnp.float32),
            pltpu.VMEM((64, SG), jnp.float32),
            pltpu.SemaphoreType.DMA,
        ],
    )
    def k(eid_hbm, scale_hbm, cnt_hbm, x_hbm,
          dest_hbm, beid_hbm, rmeta_hbm, xs_hbm, ss_hbm,
          cntv, eidv, scalev, beidv, rmetav, destv, idxA, idxB, xbuf, sbuf,
          sem):
        wid = lax.axis_index("s") * NC + lax.axis_index("c")
        base = wid * TW
        lanes = lax.iota(jnp.int32, 16)

        pltpu.sync_copy(cnt_hbm, cntv)
        pltpu.sync_copy(eid_hbm.at[pl.ds(base, TW)], eidv)
        pltpu.sync_copy(scale_hbm.at[pl.ds(base, TW)], scalev)

        tot = jnp.zeros((16,), jnp.int32)
        pref = jnp.zeros((16,), jnp.int32)
        for w in range(NW):
            row = cntv[pl.ds(w * 16, 16)]
            tot = tot + row
            pref = pref + jnp.where(w < wid, row, jnp.zeros((16,), jnp.int32))
        nblk = (tot + (BT - 1)) >> BTLOG
        blk_incl = plsc.cumsum(nblk)
        blk_excl = blk_incl - nblk
        wbase = blk_excl * BT + pref

        bused = blk_incl[NRE - 1]
        present = jnp.where(nblk > 0, 1, 0)
        rank_incl = plsc.cumsum(present)
        rank = rank_incl - present
        nruns = rank_incl[15]
        for c2 in range(NBID // 16):
            bv = lax.iota(jnp.int32, 16) + c2 * 16
            acc = jnp.zeros((16,), jnp.int32)
            rob = jnp.zeros((16,), jnp.int32)
            for e in range(NRE):
                s = blk_excl[e]
                n = nblk[e]
                inr = (bv >= s) & (bv < s + n)
                acc = jnp.where(inr, e, acc)
                rob = jnp.where(inr, rank[e], rob)
            acc = jnp.where(bv < bused, acc, -1)
            rob = jnp.where(bv < bused, rob, nruns)
            beidv[pl.ds(c2 * 16, 16)] = acc
            rmetav[pl.ds(c2 * 16, 16)] = rob
        reid = jnp.zeros((16,), jnp.int32)
        for e in range(NRE):
            reid = jnp.where((present[e] > 0) & (lanes == rank[e]), e, reid)
        reid = jnp.where(lanes == 15, nruns, reid)
        rmetav[pl.ds(NBID, 16)] = reid

        @pl.when(wid == 0)
        def _():
            pltpu.sync_copy(beidv, beid_hbm)
            pltpu.sync_copy(rmetav, rmeta_hbm)

        run = wbase
        for j in range(TW // 16):
            v = eidv[pl.ds(j * 16, 16)]
            dest16 = jnp.zeros((16,), jnp.int32)
            for e in range(NRE):
                m = v == e
                csum = plsc.cumsum(jnp.where(m, 1, 0))
                r_es = run[e]
                dest16 = jnp.where(m, r_es + csum - 1, dest16)
                run = run + jnp.where(lanes == e, csum[15], 0)
            destv[pl.ds(j * 16, 16)] = dest16
            half = idxA if j < 4 else idxB
            half[pl.ds((j % 4) * 16, 16)] = dest16
        pltpu.sync_copy(destv, dest_hbm.at[pl.ds(base, TW)])

        for half, idx in ((0, idxA), (1, idxB)):
            hb = base + half * 64
            for q in range(4):
                s16 = scalev[pl.ds(half * 64 + q * 16, 16)]
                for r in range(16):
                    sbuf[q * 16 + r, pl.ds(0, 16)] = jnp.full(
                        (16,), s16[r], jnp.float32)
            pltpu.sync_copy(x_hbm.at[pl.ds(hb, 64)], xbuf)
            pltpu.async_copy(xbuf, xs_hbm.at[idx], sem).wait()
            pltpu.async_copy(sbuf, ss_hbm.at[idx], sem).wait()

    return k(eid, scale, cnt, x)


def _fetch_run(eu, eg, ed, wup, wgt, wdn, sem, e_f, h_f, slot):
    hs = pl.ds(h_f * BH, BH)
    pltpu.make_async_copy(eu.at[e_f, :, hs], wup.at[slot], sem.at[slot]).start()
    pltpu.make_async_copy(eg.at[e_f, :, hs], wgt.at[slot], sem.at[slot]).start()
    pltpu.make_async_copy(ed.at[e_f, hs, :], wdn.at[slot], sem.at[slot]).start()


def _wait_run(eu, eg, ed, wup, wgt, wdn, sem, slot):
    hs = pl.ds(0, BH)
    pltpu.make_async_copy(eu.at[0, :, hs], wup.at[slot], sem.at[slot]).wait()
    pltpu.make_async_copy(eg.at[0, :, hs], wgt.at[slot], sem.at[slot]).wait()
    pltpu.make_async_copy(ed.at[0, hs, :], wdn.at[slot], sem.at[slot]).wait()


def _mlp_body(beid_s, rmeta_s, xs_ref, ss_ref, eu, eg, ed, su, sg, sd,
              out_ref, wup, wgt, wdn, acc_ref, sem):
    h = pl.program_id(0)
    b = pl.program_id(1)
    e = beid_s[b]
    r = rmeta_s[b]
    R = rmeta_s[NBID + 15]
    slot = lax.rem(h * R + r, 2)
    prev_r = rmeta_s[jnp.maximum(b - 1, 0)]
    first_of_run = (e >= 0) & ((b == 0) | (prev_r != r))

    @pl.when(first_of_run & (h == 0) & (b == 0))
    def _():
        _fetch_run(eu, eg, ed, wup, wgt, wdn, sem,
                   rmeta_s[NBID], 0, slot)

    @pl.when(first_of_run)
    def _():
        _wait_run(eu, eg, ed, wup, wgt, wdn, sem, slot)
        last_run = r == R - 1
        have_next = jnp.logical_not(last_run & (h == NH - 1))
        r_n = jnp.where(last_run, 0, r + 1)
        e_n = rmeta_s[NBID + r_n]
        h_n = jnp.where(last_run, h + 1, h)

        @pl.when(have_next)
        def _():
            _fetch_run(eu, eg, ed, wup, wgt, wdn, sem, e_n, h_n,
                       lax.rem(slot + 1, 2))

    @pl.when(e >= 0)
    def _():
        xb = xs_ref[...]
        up = jnp.dot(xb, wup[slot], preferred_element_type=jnp.float32)
        gt = jnp.dot(xb, wgt[slot], preferred_element_type=jnp.float32)
        a = up * (gt * jax.nn.sigmoid(gt))
        part_r = jnp.dot(a, wdn[slot], preferred_element_type=jnp.float32)
        ups = jnp.dot(xb, su[...], preferred_element_type=jnp.float32)
        gts = jnp.dot(xb, sg[...], preferred_element_type=jnp.float32)
        a_s = ups * (gts * jax.nn.sigmoid(gts))
        part_s = jnp.dot(a_s, sd[...], preferred_element_type=jnp.float32)
        sc = ss_ref[:, 0:1]
        part = sc * part_r + part_s
        sl = pl.ds(b * BT, BT)

        @pl.when(h == 0)
        def _():
            acc_ref[sl, :] = part.astype(jnp.bfloat16)

        @pl.when((h > 0) & (h < NH - 1))
        def _():
            acc_ref[sl, :] = (acc_ref[sl, :].astype(jnp.float32)
                              + part).astype(jnp.bfloat16)

        @pl.when(h == NH - 1)
        def _():
            out_ref[...] = acc_ref[sl, :].astype(jnp.float32) + part


def _grouped_mlp(beid, rmeta, xs, ss, eu, eg, ed, su, sg, sd):
    grid_spec = pltpu.PrefetchScalarGridSpec(
        num_scalar_prefetch=2,
        grid=(NH, NBR),
        in_specs=[
            pl.BlockSpec((BT, D), lambda h, b, *_: (b, 0)),
            pl.BlockSpec((BT, SG), lambda h, b, *_: (b, 0)),
            pl.BlockSpec(memory_space=pltpu.HBM),
            pl.BlockSpec(memory_space=pltpu.HBM),
            pl.BlockSpec(memory_space=pltpu.HBM),
            pl.BlockSpec((D, BH), lambda h, b, *_: (0, h)),
            pl.BlockSpec((D, BH), lambda h, b, *_: (0, h)),
            pl.BlockSpec((BH, D), lambda h, b, *_: (h, 0)),
        ],
        out_specs=pl.BlockSpec((BT, D), lambda h, b, *_: (b, 0)),
        scratch_shapes=[
            pltpu.VMEM((2, D, BH), jnp.float32),
            pltpu.VMEM((2, D, BH), jnp.float32),
            pltpu.VMEM((2, BH, D), jnp.float32),
            pltpu.VMEM((TP, D), jnp.bfloat16),
            pltpu.SemaphoreType.DMA((2,)),
        ],
    )
    return pl.pallas_call(
        _mlp_body,
        grid_spec=grid_spec,
        out_shape=jax.ShapeDtypeStruct((TP, D), jnp.float32),
        compiler_params=pltpu.CompilerParams(
            dimension_semantics=("arbitrary", "arbitrary"),
            vmem_limit_bytes=64 * 1024 * 1024),
    )(beid, rmeta, xs, ss, eu, eg, ed, su, sg, sd)


def _gather_out(rows, dest):
    mesh = plsc.VectorSubcoreMesh(core_axis_name="c", subcore_axis_name="s",
                                  num_cores=NC, num_subcores=NS)

    @functools.partial(
        pl.kernel,
        mesh=mesh,
        compiler_params=pltpu.CompilerParams(needs_layout_passes=False),
        out_type=jax.ShapeDtypeStruct((T, D), jnp.float32),
        scratch_types=[
            pltpu.VMEM((TW,), jnp.int32),
            pltpu.VMEM((64,), jnp.int32),
            pltpu.VMEM((64, D), jnp.float32),
            pltpu.SemaphoreType.DMA,
        ],
    )
    def k(rows_hbm, dest_hbm, out_hbm, destv, idx, buf, sem):
        wid = lax.axis_index("s") * NC + lax.axis_index("c")
        base = wid * TW
        pltpu.sync_copy(dest_hbm.at[pl.ds(base, TW)], destv)
        for half in range(2):
            for q in range(4):
                idx[pl.ds(q * 16, 16)] = destv[pl.ds(half * 64 + q * 16, 16)]
            pltpu.async_copy(rows_hbm.at[idx], buf, sem).wait()
            pltpu.sync_copy(buf, out_hbm.at[pl.ds(base + half * 64, 64)])

    return k(rows, dest)


def kernel(x, router_w, expert_up, expert_gate, expert_down,
           shared_up, shared_gate, shared_down):
    eid, scale, cnt = _router(x, router_w)
    dest, beid, rmeta, xs, ss = _sort_scatter(eid, scale, cnt, x)
    rows = _grouped_mlp(beid, rmeta, xs, ss, expert_up, expert_gate,
                        expert_down, shared_up, shared_gate, shared_down)
    return _gather_out(rows, dest)
